# baseline (speedup 1.0000x reference)
import jax
import jax.numpy as jnp
from jax import lax
from jax.experimental import pallas as pl
from jax.experimental.pallas import tpu as pltpu
from jax.experimental.pallas import tpu_sc as plsc
from jax._src.pallas import mpmd as _mpmd

N = 100000
D = 64
B = 16384
L = 16
NC = 2
NS = 16
NW = NC * NS
CPT = N // NW
CH = 64
CHC = 128
NGA = B // L
YST = 4096
CPAD = 3136
NGC = CPAD // L
MCAP = B + CH
PCAPR = 3200
SLOTS = 17424
TRASH = SLOTS - 1
DUMMY = CPT * 16384
NCOPY = (CPT + CHC - 1) // CHC


def _bcast_lane(v, j):
    idx = jnp.full((L, 1), j, dtype=jnp.int32)
    dn = lax.GatherDimensionNumbers(
        offset_dims=(), collapsed_slice_dims=(0,), start_index_map=(0,))
    return lax.gather(v, idx, dn, slice_sizes=(1,),
                      mode=lax.GatherScatterMode.PROMISE_IN_BOUNDS)


def _body(y_hbm, x_hbm, c_hbm, ctr_hbm, out_hbm,
          ysc, memb, counts, plist, slotmap,
          xb0, xb1, cb0, cb1, sb0, sb1, zbuf,
          ib0, ib1, jb0, jb1, ci0, ci1, ctr_buf, base_smem, sums_sh,
          gx0, gx1, sa0, sa1, gc0, gc1, gs0, gs1, ss0, ss1, zsem):
    xb = (xb0, xb1)
    cb = (cb0, cb1)
    sb = (sb0, sb1)
    ib = (ib0, ib1)
    jb = (jb0, jb1)
    ci = (ci0, ci1)
    gx = (gx0, gx1)
    sa = (sa0, sa1)
    gc = (gc0, gc1)
    gs = (gs0, gs1)
    ss = (ss0, ss1)

    iota = lax.iota(jnp.int32, L)
    lane0 = iota == 0
    zeros = jnp.zeros((L,), jnp.float32)

    sid = lax.axis_index("s")
    wid = sid * NC + lax.axis_index("c")
    lo = wid * CPT
    hi = lo + CPT

    @pl.when(sid == 0)
    def _():
        base_smem[0] = 0
    plsc.subcore_barrier()

    ctr_buf[...] = zeros
    pltpu.sync_copy(ctr_hbm, ctr_buf.at[pl.ds(0, 1)])
    ctr_b = _bcast_lane(ctr_buf[...], 0)
    r1v = 1.0 / (ctr_b + 1.0)
    a_old_v = ctr_b * r1v

    dummy_vec = jnp.full((L,), DUMMY, jnp.int32)

    def init_memb(g, _):
        memb[pl.ds(g * L, L)] = dummy_vec
        return 0
    lax.fori_loop(0, MCAP // L, init_memb, 0)

    zvec_i = jnp.zeros((L,), jnp.int32)

    def init_plist(g, _):
        plist[pl.ds(g * L, L)] = zvec_i
        return 0
    lax.fori_loop(0, PCAPR // L, init_plist, 0)

    def init_counts(g, _):
        counts[pl.ds(g * L, L)] = zeros
        return 0
    lax.fori_loop(0, CPAD // L, init_counts, 0)

    trash_vec = jnp.full((L,), TRASH, jnp.int32)

    def init_slotmap(g, _):
        slotmap[pl.ds(g * L, L)] = trash_vec
        return 0
    lax.fori_loop(0, CPAD // L, init_slotmap, 0)

    def init_zbuf(r, _):
        for seg in range(D // L):
            zbuf[r, pl.ds(seg * L, L)] = zeros
        return 0
    lax.fori_loop(0, CH, init_zbuf, 0)

    def phase_a_stage(st, off):
        pltpu.sync_copy(y_hbm.at[pl.ds(st * YST, YST)], ysc)

        def phase_a(g, off):
            yv = ysc[pl.ds(g * L, L)]
            inm = (yv >= lo) & (yv < hi)
            npop = plsc.all_reduce_population_count(inm)

            @pl.when(npop[0] != 0)
            def _():
                cl = yv - lo
                packed = cl * 16384 + (st * YST + g * L + iota)
                pos = plsc.cumsum(inm.astype(jnp.int32))
                addr = off + pos - 1
                plsc.store_scatter(memb, [addr], packed, mask=inm)
            return off + npop[0]
        return lax.fori_loop(0, YST // L, phase_a, off)
    m_cnt = lax.fori_loop(0, B // YST, phase_a_stage, jnp.int32(0))
    nch = (m_cnt + CH - 1) // CH

    ones_l0 = jnp.where(lane0, 1.0, 0.0)

    def count_grp(g, _):
        sl = memb[pl.ds(g * L, L)]
        cl = sl >> 14
        for j in range(L):
            cjb = _bcast_lane(cl, j)
            plsc.addupdate_scatter(counts, [cjb], ones_l0, mask=lane0)
        return 0
    lax.fori_loop(0, nch * (CH // L), count_grp, 0)

    def pgrp(g, off):
        cnts = counts[pl.ds(g * L, L)]
        cid = g * L + iota
        pres = (cnts > 0.0) & (cid < CPT)
        pos = plsc.cumsum(pres.astype(jnp.int32))
        addr = off + pos - 1
        plsc.store_scatter(plist, [addr], cid, mask=pres)
        return off + pos[15]
    p_cnt = lax.fori_loop(0, NGC, pgrp, jnp.int32(0))
    nchp = (p_cnt + CH - 1) // CH
    p_pad = nchp * CH

    base = plsc.fetch_and_add(base_smem.at[0], p_pad, subcore_id=0)


    def slot_grp(g, _):
        pv = plist[pl.ds(g * L, L)]
        pos = g * L + iota
        plsc.store_scatter(slotmap, [pv], base + pos, mask=pos < p_cnt)
        return 0
    lax.fori_loop(0, nchp * (CH // L), slot_grp, 0)

    def zero_issue(i, _):
        pltpu.async_copy(zbuf, sums_sh.at[pl.ds(base + i * CH, CH)], zsem)
        return 0
    lax.fori_loop(0, nchp, zero_issue, 0)

    def zero_drain(i, _):
        pltpu.make_async_copy(
            zbuf, sums_sh.at[pl.ds(base + i * CH, CH)], zsem).wait()
        return 0
    lax.fori_loop(0, nchp, zero_drain, 0)

    def acc_build(t, p):
        for g8 in range(CH // L):
            sl = memb[pl.ds(t * CH + g8 * L, L)]
            ib[p][pl.ds(g8 * L, L)] = sl & 16383
            jb[p][pl.ds(g8 * L, L)] = plsc.load_gather(slotmap, [sl >> 14])
        pltpu.async_copy(x_hbm.at[ib[p]], xb[p], gx[p])

    @pl.when(nch >= 1)
    def _():
        acc_build(jnp.int32(0), 0)

    def acc_loop(t2, _):
        for p in range(2):
            t = t2 * 2 + p

            @pl.when(t < nch)
            def _():
                pltpu.make_async_copy(x_hbm.at[ib[p]], xb[p], gx[p]).wait()
                tn = t + 1
                pn = 1 - p

                @pl.when(tn < nch)
                def _():
                    @pl.when(tn >= 2)
                    def _():
                        pltpu.make_async_copy(
                            xb[pn], sums_sh.at[jb[pn]], sa[pn]).wait()
                    acc_build(tn, pn)

                pltpu.async_copy(xb[p], sums_sh.at[jb[p]], sa[p], add=True)
        return 0
    lax.fori_loop(0, (nch + 1) // 2, acc_loop, 0)

    @pl.when(nch >= 2)
    def _():
        pltpu.make_async_copy(xb[1], sums_sh.at[jb[1]], sa[1]).wait()

    @pl.when(nch >= 1)
    def _():
        pltpu.make_async_copy(xb[0], sums_sh.at[jb[0]], sa[0]).wait()

    def upd_build(t, p):
        for g8 in range(CH // L):
            pv = plist[pl.ds(t * CH + g8 * L, L)]
            ci[p][pl.ds(g8 * L, L)] = pv + lo
            jb[p][pl.ds(g8 * L, L)] = plsc.load_gather(slotmap, [pv])
        pltpu.async_copy(out_hbm.at[ci[p]], cb[p], gc[p])
        pltpu.async_copy(sums_sh.at[jb[p]], sb[p], gs[p])

    @pl.when(nchp >= 1)
    def _():
        upd_build(jnp.int32(0), 0)

    def upd_loop(t2, _):
        for p in range(2):
            t = t2 * 2 + p

            @pl.when(t < nchp)
            def _():
                pltpu.make_async_copy(out_hbm.at[ci[p]], cb[p], gc[p]).wait()
                pltpu.make_async_copy(
                    sums_sh.at[jb[p]], sb[p], gs[p]).wait()
                tn = t + 1
                pn = 1 - p

                @pl.when(tn < nchp)
                def _():
                    @pl.when(tn >= 2)
                    def _():
                        pltpu.make_async_copy(
                            cb[pn], out_hbm.at[ci[pn]], ss[pn]).wait()
                    upd_build(tn, pn)

                for g8 in range(CH // L):
                    pv = plist[pl.ds(t * CH + g8 * L, L)]
                    cnts = plsc.load_gather(counts, [pv])
                    pres = cnts > 0.0
                    ssum = jnp.where(pres, r1v / jnp.maximum(cnts, 1.0), 0.0)
                    sold = jnp.where(pres, a_old_v, 1.0)
                    for j in range(L):
                        row = g8 * L + j
                        soj = _bcast_lane(sold, j)
                        ssj = _bcast_lane(ssum, j)
                        for seg in range(D // L):
                            cv = cb[p][row, pl.ds(seg * L, L)]
                            sv = sb[p][row, pl.ds(seg * L, L)]
                            upd = jnp.where(ssj > 0.0, sv * ssj, 0.0)
                            cb[p][row, pl.ds(seg * L, L)] = cv * soj + upd

                pltpu.async_copy(cb[p], out_hbm.at[ci[p]], ss[p])
        return 0
    lax.fori_loop(0, (nchp + 1) // 2, upd_loop, 0)

    @pl.when(nchp >= 2)
    def _():
        pltpu.make_async_copy(cb[1], out_hbm.at[ci[1]], ss[1]).wait()

    @pl.when(nchp >= 1)
    def _():
        pltpu.make_async_copy(cb[0], out_hbm.at[ci[0]], ss[0]).wait()


@jax.jit
def _run(x, y, centers, counter):
    y32 = y.astype(jnp.int32)
    mesh = plsc.VectorSubcoreMesh(core_axis_name="c", subcore_axis_name="s",
                                  num_cores=NC, num_subcores=NS)
    out = _mpmd._mpmd_map(
        [(mesh, _body)],
        out_types=jax.ShapeDtypeStruct((N, D), jnp.float32),
        input_output_aliases={2: 0},
        compiler_params=pltpu.CompilerParams(use_tc_tiling_on_sc=False,
                                             needs_layout_passes=False),
        scratch_types=[
            pltpu.VMEM((YST,), jnp.int32),
            pltpu.VMEM((MCAP,), jnp.int32),
            pltpu.VMEM((CPAD,), jnp.float32),
            pltpu.VMEM((PCAPR,), jnp.int32),
            pltpu.VMEM((CPAD,), jnp.int32),
            pltpu.VMEM((CH, D), jnp.float32),
            pltpu.VMEM((CH, D), jnp.float32),
            pltpu.VMEM((CH, D), jnp.float32),
            pltpu.VMEM((CH, D), jnp.float32),
            pltpu.VMEM((CH, D), jnp.float32),
            pltpu.VMEM((CH, D), jnp.float32),
            pltpu.VMEM((CH, D), jnp.float32),
            pltpu.VMEM((CH,), jnp.int32),
            pltpu.VMEM((CH,), jnp.int32),
            pltpu.VMEM((CH,), jnp.int32),
            pltpu.VMEM((CH,), jnp.int32),
            pltpu.VMEM((CH,), jnp.int32),
            pltpu.VMEM((CH,), jnp.int32),
            pltpu.VMEM((L,), jnp.float32),
            pltpu.SMEM((1,), jnp.int32),
            pltpu.VMEM_SHARED((SLOTS, D), jnp.float32),
            pltpu.SemaphoreType.DMA,
            pltpu.SemaphoreType.DMA,
            pltpu.SemaphoreType.DMA,
            pltpu.SemaphoreType.DMA,
            pltpu.SemaphoreType.DMA,
            pltpu.SemaphoreType.DMA,
            pltpu.SemaphoreType.DMA,
            pltpu.SemaphoreType.DMA,
            pltpu.SemaphoreType.DMA,
            pltpu.SemaphoreType.DMA,
            pltpu.SemaphoreType.DMA,
        ],
    )(y32, x, centers, counter)
    return out


def kernel(x, y, centers, counter):
    new_centers = _run(x, y, centers, counter)
    return new_centers, counter + 1.0


# speedup vs baseline: 1.0527x; 1.0527x over previous
import jax
import jax.numpy as jnp
from jax import lax
from jax.experimental import pallas as pl
from jax.experimental.pallas import tpu as pltpu
from jax.experimental.pallas import tpu_sc as plsc
from jax._src.pallas import mpmd as _mpmd

N = 100000
D = 64
B = 16384
L = 16
NC = 2
NS = 16
NW = NC * NS
CPT = N // NW
CH = 64
CHC = 128
NGA = B // L
YST = 4096
CPAD = 3136
NGC = CPAD // L
MCAP = B + CH
PCAPR = 3200
SLOTS = 17424
TRASH = SLOTS - 1
DUMMY = CPT * 16384
NCOPY = (CPT + CHC - 1) // CHC


def _bcast_lane(v, j):
    idx = jnp.full((L, 1), j, dtype=jnp.int32)
    dn = lax.GatherDimensionNumbers(
        offset_dims=(), collapsed_slice_dims=(0,), start_index_map=(0,))
    return lax.gather(v, idx, dn, slice_sizes=(1,),
                      mode=lax.GatherScatterMode.PROMISE_IN_BOUNDS)


def _body(y_hbm, x_hbm, c_hbm, ctr_hbm, out_hbm,
          ysc, memb, counts, plist, slotmap,
          xb0, xb1, cb0, cb1, sb0, sb1, zbuf,
          ib0, ib1, jb0, jb1, ci0, ci1, ctr_buf, base_smem, sums_sh,
          gx0, gx1, sa0, sa1, gc0, gc1, gs0, gs1, ss0, ss1, zsem):
    xb = (xb0, xb1)
    cb = (cb0, cb1)
    sb = (sb0, sb1)
    ib = (ib0, ib1)
    jb = (jb0, jb1)
    ci = (ci0, ci1)
    gx = (gx0, gx1)
    sa = (sa0, sa1)
    gc = (gc0, gc1)
    gs = (gs0, gs1)
    ss = (ss0, ss1)

    iota = lax.iota(jnp.int32, L)
    lane0 = iota == 0
    zeros = jnp.zeros((L,), jnp.float32)

    sid = lax.axis_index("s")
    wid = sid * NC + lax.axis_index("c")
    lo = wid * CPT
    hi = lo + CPT

    @pl.when(sid == 0)
    def _():
        base_smem[0] = 0
    plsc.subcore_barrier()

    ctr_buf[...] = zeros
    pltpu.sync_copy(ctr_hbm, ctr_buf.at[pl.ds(0, 1)])
    ctr_b = _bcast_lane(ctr_buf[...], 0)
    r1v = 1.0 / (ctr_b + 1.0)
    a_old_v = ctr_b * r1v

    dummy_vec = jnp.full((L,), DUMMY, jnp.int32)

    def init_memb(g, _):
        memb[pl.ds(g * L, L)] = dummy_vec
        return 0
    lax.fori_loop(0, MCAP // L, init_memb, 0)

    zvec_i = jnp.zeros((L,), jnp.int32)

    def init_plist(g, _):
        plist[pl.ds(g * L, L)] = zvec_i
        return 0
    lax.fori_loop(0, PCAPR // L, init_plist, 0)

    def init_counts(g, _):
        counts[pl.ds(g * L, L)] = zeros
        return 0
    lax.fori_loop(0, CPAD // L, init_counts, 0)

    trash_vec = jnp.full((L,), TRASH, jnp.int32)

    def init_slotmap(g, _):
        slotmap[pl.ds(g * L, L)] = trash_vec
        return 0
    lax.fori_loop(0, CPAD // L, init_slotmap, 0)

    def init_zbuf(r, _):
        for seg in range(D // L):
            zbuf[r, pl.ds(seg * L, L)] = zeros
        return 0
    lax.fori_loop(0, CH, init_zbuf, 0)

    def phase_a_stage(st, off):
        pltpu.sync_copy(y_hbm.at[pl.ds(st * YST, YST)], ysc)

        def phase_a(g, off):
            yv = ysc[pl.ds(g * L, L)]
            inm = (yv >= lo) & (yv < hi)
            cl = yv - lo
            packed = cl * 16384 + (st * YST + g * L + iota)
            pos = plsc.cumsum(inm.astype(jnp.int32))
            addr = off + pos - 1
            plsc.store_scatter(memb, [addr], packed, mask=inm)
            return off + pos[15]
        return lax.fori_loop(0, YST // L, phase_a, off)
    m_cnt = lax.fori_loop(0, B // YST, phase_a_stage, jnp.int32(0))
    nch = (m_cnt + CH - 1) // CH

    ones_l0 = jnp.where(lane0, 1.0, 0.0)

    def count_grp(g, _):
        sl = memb[pl.ds(g * L, L)]
        cl = sl >> 14
        for j in range(L):
            cjb = _bcast_lane(cl, j)
            plsc.addupdate_scatter(counts, [cjb], ones_l0, mask=lane0)
        return 0
    lax.fori_loop(0, nch * (CH // L), count_grp, 0)

    def pgrp(g, off):
        cnts = counts[pl.ds(g * L, L)]
        cid = g * L + iota
        pres = (cnts > 0.0) & (cid < CPT)
        pos = plsc.cumsum(pres.astype(jnp.int32))
        addr = off + pos - 1
        plsc.store_scatter(plist, [addr], cid, mask=pres)
        return off + pos[15]
    p_cnt = lax.fori_loop(0, NGC, pgrp, jnp.int32(0))
    nchp = (p_cnt + CH - 1) // CH
    p_pad = nchp * CH

    base = plsc.fetch_and_add(base_smem.at[0], p_pad, subcore_id=0)


    def slot_grp(g, _):
        pv = plist[pl.ds(g * L, L)]
        pos = g * L + iota
        plsc.store_scatter(slotmap, [pv], base + pos, mask=pos < p_cnt)
        return 0
    lax.fori_loop(0, nchp * (CH // L), slot_grp, 0)

    def zero_issue(i, _):
        pltpu.async_copy(zbuf, sums_sh.at[pl.ds(base + i * CH, CH)], zsem)
        return 0
    lax.fori_loop(0, nchp, zero_issue, 0)

    def zero_drain(i, _):
        pltpu.make_async_copy(
            zbuf, sums_sh.at[pl.ds(base + i * CH, CH)], zsem).wait()
        return 0
    lax.fori_loop(0, nchp, zero_drain, 0)

    def acc_build(t, p):
        for g8 in range(CH // L):
            sl = memb[pl.ds(t * CH + g8 * L, L)]
            ib[p][pl.ds(g8 * L, L)] = sl & 16383
            jb[p][pl.ds(g8 * L, L)] = plsc.load_gather(slotmap, [sl >> 14])
        pltpu.async_copy(x_hbm.at[ib[p]], xb[p], gx[p])

    @pl.when(nch >= 1)
    def _():
        acc_build(jnp.int32(0), 0)

    def acc_loop(t2, _):
        for p in range(2):
            t = t2 * 2 + p

            @pl.when(t < nch)
            def _():
                pltpu.make_async_copy(x_hbm.at[ib[p]], xb[p], gx[p]).wait()
                tn = t + 1
                pn = 1 - p

                @pl.when(tn < nch)
                def _():
                    @pl.when(tn >= 2)
                    def _():
                        pltpu.make_async_copy(
                            xb[pn], sums_sh.at[jb[pn]], sa[pn]).wait()
                    acc_build(tn, pn)

                pltpu.async_copy(xb[p], sums_sh.at[jb[p]], sa[p], add=True)
        return 0
    lax.fori_loop(0, (nch + 1) // 2, acc_loop, 0)

    @pl.when(nch >= 2)
    def _():
        pltpu.make_async_copy(xb[1], sums_sh.at[jb[1]], sa[1]).wait()

    @pl.when(nch >= 1)
    def _():
        pltpu.make_async_copy(xb[0], sums_sh.at[jb[0]], sa[0]).wait()

    def upd_build(t, p):
        for g8 in range(CH // L):
            pv = plist[pl.ds(t * CH + g8 * L, L)]
            ci[p][pl.ds(g8 * L, L)] = pv + lo
            jb[p][pl.ds(g8 * L, L)] = plsc.load_gather(slotmap, [pv])
        pltpu.async_copy(out_hbm.at[ci[p]], cb[p], gc[p])
        pltpu.async_copy(sums_sh.at[jb[p]], sb[p], gs[p])

    @pl.when(nchp >= 1)
    def _():
        upd_build(jnp.int32(0), 0)

    def upd_loop(t2, _):
        for p in range(2):
            t = t2 * 2 + p

            @pl.when(t < nchp)
            def _():
                pltpu.make_async_copy(out_hbm.at[ci[p]], cb[p], gc[p]).wait()
                pltpu.make_async_copy(
                    sums_sh.at[jb[p]], sb[p], gs[p]).wait()
                tn = t + 1
                pn = 1 - p

                @pl.when(tn < nchp)
                def _():
                    @pl.when(tn >= 2)
                    def _():
                        pltpu.make_async_copy(
                            cb[pn], out_hbm.at[ci[pn]], ss[pn]).wait()
                    upd_build(tn, pn)

                for g8 in range(CH // L):
                    pv = plist[pl.ds(t * CH + g8 * L, L)]
                    cnts = plsc.load_gather(counts, [pv])
                    pres = cnts > 0.0
                    ssum = jnp.where(pres, r1v / jnp.maximum(cnts, 1.0), 0.0)
                    sold = jnp.where(pres, a_old_v, 1.0)
                    for j in range(L):
                        row = g8 * L + j
                        soj = _bcast_lane(sold, j)
                        ssj = _bcast_lane(ssum, j)
                        for seg in range(D // L):
                            cv = cb[p][row, pl.ds(seg * L, L)]
                            sv = sb[p][row, pl.ds(seg * L, L)]
                            upd = jnp.where(ssj > 0.0, sv * ssj, 0.0)
                            cb[p][row, pl.ds(seg * L, L)] = cv * soj + upd

                pltpu.async_copy(cb[p], out_hbm.at[ci[p]], ss[p])
        return 0
    lax.fori_loop(0, (nchp + 1) // 2, upd_loop, 0)

    @pl.when(nchp >= 2)
    def _():
        pltpu.make_async_copy(cb[1], out_hbm.at[ci[1]], ss[1]).wait()

    @pl.when(nchp >= 1)
    def _():
        pltpu.make_async_copy(cb[0], out_hbm.at[ci[0]], ss[0]).wait()


@jax.jit
def _run(x, y, centers, counter):
    y32 = y.astype(jnp.int32)
    mesh = plsc.VectorSubcoreMesh(core_axis_name="c", subcore_axis_name="s",
                                  num_cores=NC, num_subcores=NS)
    out = _mpmd._mpmd_map(
        [(mesh, _body)],
        out_types=jax.ShapeDtypeStruct((N, D), jnp.float32),
        input_output_aliases={2: 0},
        compiler_params=pltpu.CompilerParams(use_tc_tiling_on_sc=False,
                                             needs_layout_passes=False),
        scratch_types=[
            pltpu.VMEM((YST,), jnp.int32),
            pltpu.VMEM((MCAP,), jnp.int32),
            pltpu.VMEM((CPAD,), jnp.float32),
            pltpu.VMEM((PCAPR,), jnp.int32),
            pltpu.VMEM((CPAD,), jnp.int32),
            pltpu.VMEM((CH, D), jnp.float32),
            pltpu.VMEM((CH, D), jnp.float32),
            pltpu.VMEM((CH, D), jnp.float32),
            pltpu.VMEM((CH, D), jnp.float32),
            pltpu.VMEM((CH, D), jnp.float32),
            pltpu.VMEM((CH, D), jnp.float32),
            pltpu.VMEM((CH, D), jnp.float32),
            pltpu.VMEM((CH,), jnp.int32),
            pltpu.VMEM((CH,), jnp.int32),
            pltpu.VMEM((CH,), jnp.int32),
            pltpu.VMEM((CH,), jnp.int32),
            pltpu.VMEM((CH,), jnp.int32),
            pltpu.VMEM((CH,), jnp.int32),
            pltpu.VMEM((L,), jnp.float32),
            pltpu.SMEM((1,), jnp.int32),
            pltpu.VMEM_SHARED((SLOTS, D), jnp.float32),
            pltpu.SemaphoreType.DMA,
            pltpu.SemaphoreType.DMA,
            pltpu.SemaphoreType.DMA,
            pltpu.SemaphoreType.DMA,
            pltpu.SemaphoreType.DMA,
            pltpu.SemaphoreType.DMA,
            pltpu.SemaphoreType.DMA,
            pltpu.SemaphoreType.DMA,
            pltpu.SemaphoreType.DMA,
            pltpu.SemaphoreType.DMA,
            pltpu.SemaphoreType.DMA,
        ],
    )(y32, x, centers, counter)
    return out


def kernel(x, y, centers, counter):
    new_centers = _run(x, y, centers, counter)
    return new_centers, counter + 1.0
